# baseline (speedup 1.0000x reference)
import functools

import jax
import jax.numpy as jnp
from jax import lax
from jax.experimental import pallas as pl
from jax.experimental.pallas import tpu as pltpu, tpu_sc as plsc


def _make_lookup(V, D, B):
    info = plsc.get_sparse_core_info()
    NC = info.num_cores
    half = B // NC
    CH = 1024
    mesh = plsc.ScalarSubcoreMesh(axis_name="c")

    @functools.partial(
        pl.kernel,
        mesh=mesh,
        out_type=jax.ShapeDtypeStruct((B, D), jnp.float32),
        scratch_types=[
            pltpu.SMEM((CH,), jnp.int32),
            pltpu.VMEM_SHARED((half, D), jnp.float32),
            pltpu.SemaphoreType.DMA,
            pltpu.SemaphoreType.DMA,
        ],
    )
    def k(table_hbm, idx_hbm, out_hbm, idx_s, rows_sh, sem, osem):
        cid = lax.axis_index("c")
        base = cid * half

        for chunk in range(half // CH):
            pltpu.sync_copy(idx_hbm.at[pl.ds(base + chunk * CH, CH)], idx_s)

            def fire(r, carry):
                pltpu.async_copy(
                    table_hbm.at[pl.ds(idx_s[r], 1)],
                    rows_sh.at[pl.ds(chunk * CH + r, 1)],
                    sem,
                )
                return carry

            lax.fori_loop(0, CH, fire, 0)

        pltpu.make_async_copy(
            table_hbm.at[pl.ds(0, half)], rows_sh, sem
        ).wait()
        pltpu.async_copy(rows_sh, out_hbm.at[pl.ds(base, half)], osem).wait()

    return k


def kernel(labels, embed_table):
    B = labels.shape[0]
    V, D = embed_table.shape
    lookup = _make_lookup(V, D, B)
    return lookup(embed_table, labels.astype(jnp.int32))


# speedup vs baseline: 1.1303x; 1.1303x over previous
import functools

import jax
import jax.numpy as jnp
from jax import lax
from jax.experimental import pallas as pl
from jax.experimental.pallas import tpu as pltpu, tpu_sc as plsc


def _make_lookup(V, D, B):
    info = plsc.get_sparse_core_info()
    NC, NS, L = info.num_cores, info.num_subcores, info.num_lanes
    NW = NC * NS
    assert B % (8 * NW) == 0
    b_per_w = B // NW
    mesh = plsc.VectorSubcoreMesh(core_axis_name="c", subcore_axis_name="s")

    @functools.partial(
        pl.kernel,
        mesh=mesh,
        out_type=jax.ShapeDtypeStruct((B, D), jnp.float32),
        scratch_types=[
            pltpu.VMEM((b_per_w,), jnp.int32),
            pltpu.VMEM((b_per_w, D), jnp.float32),
            pltpu.SemaphoreType.DMA,
        ],
    )
    def k(table_hbm, idx_hbm, out_hbm, idx_v, rows_v, sem):
        wid = lax.axis_index("s") * NC + lax.axis_index("c")
        base = wid * b_per_w
        pltpu.sync_copy(idx_hbm.at[pl.ds(base, b_per_w)], idx_v)

        def fire(c, carry):
            iv = idx_v[pl.ds(c * L, L)]
            for u in range(L):
                r = c * L + u
                pltpu.async_copy(table_hbm.at[iv[u]], rows_v.at[r], sem)
            return carry

        lax.fori_loop(0, b_per_w // L, fire, 0)
        pltpu.make_async_copy(
            table_hbm.at[pl.ds(0, b_per_w)], rows_v, sem
        ).wait()
        pltpu.sync_copy(rows_v, out_hbm.at[pl.ds(base, b_per_w)])

    return k


def kernel(labels, embed_table):
    B = labels.shape[0]
    V, D = embed_table.shape
    lookup = _make_lookup(V, D, B)
    return lookup(embed_table, labels.astype(jnp.int32))


# speedup vs baseline: 2.0884x; 1.8476x over previous
import functools

import jax
import jax.numpy as jnp
from jax import lax
from jax.experimental import pallas as pl
from jax.experimental.pallas import tpu as pltpu, tpu_sc as plsc


def _make_lookup(V, D, B):
    info = plsc.get_sparse_core_info()
    NC, NS, L = info.num_cores, info.num_subcores, info.num_lanes
    NW = NC * NS
    assert B % (8 * NW) == 0 and D == 32
    b_per_w = B // NW
    C = 8
    mesh = plsc.VectorSubcoreMesh(core_axis_name="c", subcore_axis_name="s")

    @functools.partial(
        pl.kernel,
        mesh=mesh,
        out_type=jax.ShapeDtypeStruct((D, B), jnp.float32),
        scratch_types=[
            pltpu.VMEM((b_per_w,), jnp.int32),
            pltpu.VMEM((C * D, 128), jnp.float32),
            pltpu.VMEM((D, b_per_w), jnp.float32),
            pltpu.SemaphoreType.DMA,
        ],
        compiler_params=pltpu.CompilerParams(needs_layout_passes=False),
    )
    def k(tt_hbm, idx_hbm, out_t_hbm, idx_v, blk, out_t, sem):
        wid = lax.axis_index("s") * NC + lax.axis_index("c")
        base = wid * b_per_w
        pltpu.sync_copy(idx_hbm.at[pl.ds(base, b_per_w)], idx_v)

        iota = lax.iota(jnp.int32, L)

        def chunk(p, carry):
            iv = idx_v[pl.ds(p * 2 * C, L)]
            for half in range(2):
                for j in range(C):
                    lab = iv[half * C + j]
                    q = lax.shift_left(
                        lax.shift_right_logical(lab, 7), 7
                    )
                    pltpu.async_copy(
                        tt_hbm.at[:, pl.ds(pl.multiple_of(q, 128), 128)],
                        blk.at[pl.ds(j * D, D)],
                        sem,
                    )
                for j in range(C):
                    pltpu.make_async_copy(
                        tt_hbm.at[:, pl.ds(0, 128)],
                        blk.at[pl.ds(j * D, D)],
                        sem,
                    ).wait()
                for j in range(C):
                    lab = iv[half * C + j]
                    rem = jnp.full((L,), lab & 127, jnp.int32)
                    col = p * 2 * C + half * C + j
                    for h in range(D // L):
                        rows = (j * D + h * L) + iota
                        vals = plsc.load_gather(blk, [rows, rem])
                        plsc.store_scatter(
                            out_t,
                            [h * L + iota, jnp.full((L,), col, jnp.int32)],
                            vals,
                        )
            return carry

        lax.fori_loop(0, b_per_w // (2 * C), chunk, 0)
        pltpu.sync_copy(
            out_t, out_t_hbm.at[:, pl.ds(pl.multiple_of(base, 128), b_per_w)]
        )

    return k


def kernel(labels, embed_table):
    B = labels.shape[0]
    V, D = embed_table.shape
    lookup = _make_lookup(V, D, B)
    out_t = lookup(embed_table.T, labels.astype(jnp.int32))
    return out_t.T


# speedup vs baseline: 2.5988x; 1.2444x over previous
import functools

import jax
import jax.numpy as jnp
from jax import lax
from jax.experimental import pallas as pl
from jax.experimental.pallas import tpu as pltpu, tpu_sc as plsc


def _make_lookup(V, D, B):
    info = plsc.get_sparse_core_info()
    NC, NS, L = info.num_cores, info.num_subcores, info.num_lanes
    NW = NC * NS
    assert B % (8 * NW) == 0 and D == 32
    b_per_w = B // NW
    C = 8
    n_pairs = b_per_w // (2 * C)
    mesh = plsc.VectorSubcoreMesh(core_axis_name="c", subcore_axis_name="s")

    @functools.partial(
        pl.kernel,
        mesh=mesh,
        out_type=jax.ShapeDtypeStruct((D, B), jnp.float32),
        scratch_types=[
            pltpu.VMEM((b_per_w,), jnp.int32),
            pltpu.VMEM((C * D, 128), jnp.float32),
            pltpu.VMEM((C * D, 128), jnp.float32),
            pltpu.VMEM((D, b_per_w), jnp.float32),
            pltpu.SemaphoreType.DMA,
            pltpu.SemaphoreType.DMA,
        ],
        compiler_params=pltpu.CompilerParams(needs_layout_passes=False),
    )
    def k(tt_hbm, idx_hbm, out_t_hbm, idx_v, blk0, blk1, out_t, sem0, sem1):
        wid = lax.axis_index("s") * NC + lax.axis_index("c")
        base = wid * b_per_w
        pltpu.sync_copy(idx_hbm.at[pl.ds(base, b_per_w)], idx_v)

        iota = lax.iota(jnp.int32, L)

        def fire(iv, half, blk, sem):
            for j in range(C):
                lab = iv[half * C + j]
                q = lax.shift_left(lax.shift_right_logical(lab, 7), 7)
                pltpu.async_copy(
                    tt_hbm.at[:, pl.ds(pl.multiple_of(q, 128), 128)],
                    blk.at[pl.ds(j * D, D)],
                    sem,
                )

        def drain_extract(iv, half, chunk_id, blk, sem):
            for j in range(C):
                pltpu.make_async_copy(
                    tt_hbm.at[:, pl.ds(0, 128)], blk.at[pl.ds(j * D, D)], sem
                ).wait()
            for j in range(C):
                lab = iv[half * C + j]
                rem = jnp.full((L,), lab & 127, jnp.int32)
                col = chunk_id * C + j
                for h in range(D // L):
                    rows = (j * D + h * L) + iota
                    vals = plsc.load_gather(blk, [rows, rem])
                    plsc.store_scatter(
                        out_t,
                        [h * L + iota, jnp.full((L,), col, jnp.int32)],
                        vals,
                    )

        iv0 = idx_v[pl.ds(0, L)]
        fire(iv0, 0, blk0, sem0)

        def pair(p, carry):
            iv = idx_v[pl.ds(p * 2 * C, L)]
            fire(iv, 1, blk1, sem1)
            drain_extract(iv, 0, p * 2, blk0, sem0)

            @pl.when(p < n_pairs - 1)
            def _fire_next():
                ivn = idx_v[pl.ds((p + 1) * 2 * C, L)]
                fire(ivn, 0, blk0, sem0)

            drain_extract(iv, 1, p * 2 + 1, blk1, sem1)
            return carry

        lax.fori_loop(0, n_pairs, pair, 0)
        pltpu.sync_copy(
            out_t, out_t_hbm.at[:, pl.ds(pl.multiple_of(base, 128), b_per_w)]
        )

    return k


def kernel(labels, embed_table):
    B = labels.shape[0]
    V, D = embed_table.shape
    lookup = _make_lookup(V, D, B)
    out_t = lookup(embed_table.T, labels.astype(jnp.int32))
    return out_t.T
